# Initial kernel scaffold; baseline (speedup 1.0000x reference)
import jax
import jax.numpy as jnp
from jax.experimental import pallas as pl


def kernel(x, edge_index, W1, b1, W2, b2):
    raise NotImplementedError("write your pallas kernel here")


# speedup vs baseline: 10.7298x; 10.7298x over previous
import functools

import jax
import jax.numpy as jnp
from jax import lax
from jax.experimental import pallas as pl
from jax.experimental.pallas import tpu as pltpu
from jax.experimental.pallas import tpu_sc as plsc

N = 10000
E = 160000
D = 256
NC, NS, L = 2, 16, 16

DEG_PAD = 10240
DEG_TILE = DEG_PAD // NS
EPT_D = E // NS
NQ_D = EPT_D // 128

HALF = 5120
ACC_ROWS = HALF + 8
DUMMY = HALF
EPT = E // NS
NQ = EPT // 128
CK = 48
CLEN = EPT + 128
ROWS_T = HALF // NS
OUT_CK = 40
OUT_CH = ROWS_T // OUT_CK

BLK = 2000


@functools.cache
def _sc_kernels():
    mesh = plsc.VectorSubcoreMesh(core_axis_name="c", subcore_axis_name="s",
                                  num_cores=NC, num_subcores=NS)

    @functools.partial(
        pl.kernel,
        out_type=jax.ShapeDtypeStruct((DEG_PAD,), jnp.float32),
        mesh=mesh,
        compiler_params=pltpu.CompilerParams(needs_layout_passes=False),
        scratch_types=[
            pltpu.VMEM((128,), jnp.int32),
            pltpu.VMEM((128,), jnp.int32),
            pltpu.VMEM((16,), jnp.int32),
            pltpu.VMEM((128,), jnp.float32),
            pltpu.VMEM((DEG_TILE,), jnp.float32),
            pltpu.VMEM_SHARED((DEG_PAD,), jnp.float32),
            pltpu.SemaphoreType.DMA,
            pltpu.SemaphoreType.DMA,
        ],
    )
    def _deg_kernel(ei_hbm, deg_hbm, idx0, idx1, idxt, ones, zbuf, spdeg, sem0, sem1):
        c = lax.axis_index("c")
        s = lax.axis_index("s")

        @pl.when(c == 0)
        def _():
            zero16 = jnp.zeros((L,), jnp.float32)
            one16 = jnp.ones((L,), jnp.float32)
            ebase = s * EPT_D

            @pl.loop(0, 128 // L)
            def _(i):
                ones[pl.ds(i * L, L)] = one16

            @pl.loop(0, DEG_TILE // L)
            def _(i):
                zbuf[pl.ds(i * L, L)] = zero16

            pltpu.sync_copy(zbuf, spdeg.at[pl.ds(s * DEG_TILE, DEG_TILE)])
            plsc.subcore_barrier()

            @pl.loop(0, NQ_D // 2)
            def _(p):
                for par, (ib, sm) in enumerate(((idx0, sem0), (idx1, sem1))):
                    k = p * 2 + par

                    @pl.when(p >= 1)
                    def _():
                        pltpu.make_async_copy(ones, spdeg.at[pl.ds(0, 128)], sm).wait()

                    pltpu.sync_copy(ei_hbm.at[pl.ds(E + ebase + k * 128, 128)], ib)
                    pltpu.async_copy(ones, spdeg.at[ib], sm, add=True)

            for sm in (sem0, sem1):
                pltpu.make_async_copy(ones, spdeg.at[pl.ds(0, 128)], sm).wait()

            pltpu.sync_copy(ei_hbm.at[pl.ds(E + ebase + NQ_D * 128, L)], idxt)
            pltpu.async_copy(ones.at[pl.ds(0, L)], spdeg.at[idxt], sem0, add=True)
            pltpu.make_async_copy(ones.at[pl.ds(0, L)], spdeg.at[pl.ds(0, L)], sem0).wait()

            plsc.subcore_barrier()
            pltpu.sync_copy(spdeg.at[pl.ds(s * DEG_TILE, DEG_TILE)], zbuf)
            pltpu.sync_copy(zbuf, deg_hbm.at[pl.ds(s * DEG_TILE, DEG_TILE)])

    @functools.partial(
        pl.kernel,
        out_type=jax.ShapeDtypeStruct((NC * HALF, 2, 128), jnp.float32),
        mesh=mesh,
        compiler_params=pltpu.CompilerParams(needs_layout_passes=False),
        scratch_types=[
            pltpu.VMEM((128,), jnp.int32),
            pltpu.VMEM((128,), jnp.int32),
            pltpu.VMEM((128,), jnp.int32),
            pltpu.VMEM((128,), jnp.int32),
            pltpu.VMEM((CLEN,), jnp.int32),
            pltpu.VMEM((CLEN,), jnp.int32),
            pltpu.VMEM((CK,), jnp.int32),
            pltpu.VMEM((CK,), jnp.int32),
            pltpu.VMEM((CK, 2, 128), jnp.float32),
            pltpu.VMEM((CK, 2, 128), jnp.float32),
            pltpu.VMEM_SHARED((ACC_ROWS, 2, 128), jnp.float32),
            pltpu.SemaphoreType.DMA,
            pltpu.SemaphoreType.DMA,
            pltpu.SemaphoreType.DMA,
            pltpu.SemaphoreType.DMA,
            pltpu.SemaphoreType.DMA,
            pltpu.SemaphoreType.DMA,
        ],
    )
    def _edge_scatter(t_hbm, ei_hbm, out_hbm,
                      sa0, sa1, da0, da1, csrc, cldst, ld0, ld1, stg0, stg1, accS,
                      isem0, isem1, gsem0, gsem1, ssem0, ssem1):
        c = lax.axis_index("c")
        s = lax.axis_index("s")
        base = c * HALF
        ebase = s * EPT
        zero16 = jnp.zeros((L,), jnp.float32)
        izero16 = jnp.zeros((L,), jnp.int32)
        idummy16 = jnp.full((L,), DUMMY, jnp.int32)
        rbase = s * ROWS_T
        iring = ((sa0, da0, isem0), (sa1, da1, isem1))
        ring = ((ld0, stg0, gsem0, ssem0), (ld1, stg1, gsem1, ssem1))

        @pl.loop(0, CK)
        def _(r):
            for u in range(2):
                @pl.loop(0, 128 // L)
                def _(k):
                    stg0[r, u, pl.ds(k * L, L)] = zero16

        @pl.loop(0, OUT_CH)
        def _(i):
            g0 = base + rbase + i * OUT_CK

            @pl.when(g0 < N)
            def _():
                pltpu.sync_copy(t_hbm.at[pl.ds(g0, OUT_CK)],
                                accS.at[pl.ds(rbase + i * OUT_CK, OUT_CK)])

            @pl.when(g0 >= N)
            def _():
                pltpu.sync_copy(stg0.at[pl.ds(0, OUT_CK)],
                                accS.at[pl.ds(rbase + i * OUT_CK, OUT_CK)])

        @pl.loop(0, CLEN // L)
        def _(i):
            csrc[pl.ds(i * L, L)] = izero16
            cldst[pl.ds(i * L, L)] = idummy16

        plsc.subcore_barrier()

        def _issue_idx(q, par):
            sb, db, ism = iring[par]
            pltpu.async_copy(ei_hbm.at[pl.ds(ebase + q * 128, 128)], sb, ism)
            pltpu.async_copy(ei_hbm.at[pl.ds(E + ebase + q * 128, 128)], db, ism)

        for par in range(2):
            _issue_idx(par, par)

        def _compact16(sv, lv, cnt):
            ok = (lv >= 0) & (lv < HALF)
            plsc.store_compressed(csrc.at[pl.ds(cnt, L)], sv, mask=ok)
            plsc.store_compressed(cldst.at[pl.ds(cnt, L)], lv, mask=ok)
            pc = plsc.all_reduce_population_count(ok)
            return cnt + pc[0]

        def _cgroup(Gq, cnt):
            for par in range(2):
                q = Gq * 2 + par
                sb, db, ism = iring[par]
                pltpu.make_async_copy(ei_hbm.at[pl.ds(0, 128)], sb, ism).wait()
                pltpu.make_async_copy(ei_hbm.at[pl.ds(0, 128)], db, ism).wait()

                def _sub(j, cnt):
                    return _compact16(sb[pl.ds(j * L, L)],
                                      db[pl.ds(j * L, L)] - base, cnt)

                cnt = pl.loop(0, 128 // L, init_carry=cnt)(_sub)
                nq = q + 2

                @pl.when(nq < NQ)
                def _():
                    _issue_idx(nq, par)
            return cnt

        cnt = pl.loop(0, NQ // 2, init_carry=jnp.int32(0))(_cgroup)

        pltpu.sync_copy(ei_hbm.at[pl.ds(ebase + NQ * 128, L)], sa0.at[pl.ds(0, L)])
        pltpu.sync_copy(ei_hbm.at[pl.ds(E + ebase + NQ * 128, L)], da0.at[pl.ds(0, L)])
        cnt = _compact16(sa0[pl.ds(0, L)], da0[pl.ds(0, L)] - base, cnt)

        k0 = (cnt // L) * L
        lanes = lax.iota(jnp.int32, L)
        cur = cldst[pl.ds(k0, L)]
        cldst[pl.ds(k0, L)] = jnp.where(lanes >= (cnt - k0), DUMMY, cur)

        @pl.loop(1, 7)
        def _(i):
            cldst[pl.ds(k0 + i * L, L)] = idummy16

        nch = 2 * ((cnt + 2 * CK - 1) // (2 * CK))

        @pl.loop(0, nch // 2)
        def _(G):
            for b in range(2):
                g = G * 2 + b
                lb, stg, gsm, ssm = ring[b]
                lbo, stgo, gsmo, ssmo = ring[1 - b]

                @pl.when(G >= 1)
                def _():
                    pltpu.make_async_copy(stg, accS.at[pl.ds(0, CK)], ssm).wait()

                for oo in range(CK // L):
                    lb[pl.ds(oo * L, L)] = cldst[pl.ds(g * CK + oo * L, L)]

                pltpu.async_copy(t_hbm.at[csrc.at[pl.ds(g * CK, CK)]], stg, gsm)

                def _drain_prev():
                    pltpu.make_async_copy(t_hbm.at[pl.ds(0, CK)], stgo, gsmo).wait()
                    pltpu.async_copy(stgo, accS.at[lbo], ssmo, add=True)

                if b == 0:
                    @pl.when(G >= 1)
                    def _():
                        _drain_prev()
                else:
                    _drain_prev()

        @pl.when(nch > 0)
        def _():
            pltpu.make_async_copy(t_hbm.at[pl.ds(0, CK)], stg1, gsem1).wait()
            pltpu.async_copy(stg1, accS.at[ld1], ssem1, add=True)
            pltpu.make_async_copy(stg0, accS.at[pl.ds(0, CK)], ssem0).wait()
            pltpu.make_async_copy(stg1, accS.at[pl.ds(0, CK)], ssem1).wait()

        plsc.subcore_barrier()

        @pl.loop(0, OUT_CH)
        def _(i):
            pltpu.sync_copy(accS.at[pl.ds(rbase + i * OUT_CK, OUT_CK)],
                            stg0.at[pl.ds(0, OUT_CK)])
            pltpu.sync_copy(stg0.at[pl.ds(0, OUT_CK)],
                            out_hbm.at[pl.ds(base + rbase + i * OUT_CK, OUT_CK)])

    return _deg_kernel, _edge_scatter


def _matmul(x, W):
    def body(x_ref, w_ref, o_ref):
        o_ref[...] = jnp.dot(x_ref[...], w_ref[...], preferred_element_type=jnp.float32)

    return pl.pallas_call(
        body,
        grid=(N // BLK,),
        in_specs=[
            pl.BlockSpec((BLK, D), lambda i: (i, 0)),
            pl.BlockSpec((D, D), lambda i: (0, 0)),
        ],
        out_specs=pl.BlockSpec((BLK, D), lambda i: (i, 0)),
        out_shape=jax.ShapeDtypeStruct((N, D), jnp.float32),
    )(x, W)


def _scale(g, deg_col):
    def body(g_ref, dg_ref, o_ref):
        dv = lax.rsqrt(dg_ref[...] + 1.0)
        o_ref[...] = g_ref[...] * dv

    return pl.pallas_call(
        body,
        grid=(N // BLK,),
        in_specs=[
            pl.BlockSpec((BLK, D), lambda i: (i, 0)),
            pl.BlockSpec((BLK, 1), lambda i: (i, 0)),
        ],
        out_specs=pl.BlockSpec((BLK, D), lambda i: (i, 0)),
        out_shape=jax.ShapeDtypeStruct((N, D), jnp.float32),
    )(g, deg_col)


def _mid_layer(acc1, W2, b1_row, deg_col):
    def body(a_ref, w_ref, b_ref, dg_ref, o_ref):
        dv = lax.rsqrt(dg_ref[...] + 1.0)
        h = jnp.maximum(a_ref[...] * dv + b_ref[...], 0.0)
        o_ref[...] = jnp.dot(h, w_ref[...], preferred_element_type=jnp.float32) * dv

    return pl.pallas_call(
        body,
        grid=(N // BLK,),
        in_specs=[
            pl.BlockSpec((BLK, D), lambda i: (i, 0)),
            pl.BlockSpec((D, D), lambda i: (0, 0)),
            pl.BlockSpec((1, D), lambda i: (0, 0)),
            pl.BlockSpec((BLK, 1), lambda i: (i, 0)),
        ],
        out_specs=pl.BlockSpec((BLK, D), lambda i: (i, 0)),
        out_shape=jax.ShapeDtypeStruct((N, D), jnp.float32),
    )(acc1, W2, b1_row, deg_col)


def _final_layer(acc2, b2_row, deg_col):
    def body(a_ref, b_ref, dg_ref, o_ref):
        dv = lax.rsqrt(dg_ref[...] + 1.0)
        o = a_ref[...] * dv + b_ref[...]
        m = jnp.max(o, axis=1, keepdims=True)
        e = jnp.exp(o - m)
        lse = jnp.log(jnp.sum(e, axis=1, keepdims=True))
        o_ref[...] = o - m - lse

    return pl.pallas_call(
        body,
        grid=(N // BLK,),
        in_specs=[
            pl.BlockSpec((BLK, D), lambda i: (i, 0)),
            pl.BlockSpec((1, D), lambda i: (0, 0)),
            pl.BlockSpec((BLK, 1), lambda i: (i, 0)),
        ],
        out_specs=pl.BlockSpec((BLK, D), lambda i: (i, 0)),
        out_shape=jax.ShapeDtypeStruct((N, D), jnp.float32),
    )(acc2, b2_row, deg_col)


def kernel(x, edge_index, W1, b1, W2, b2):
    deg_kernel, edge_scatter = _sc_kernels()

    ei = edge_index.reshape(2 * E)
    deg = deg_kernel(ei)
    g1 = _matmul(x, W1)
    deg_col = deg[:N].reshape(N, 1)

    t1 = _scale(g1, deg_col)
    acc1 = edge_scatter(t1.reshape(N, 2, 128), ei).reshape(NC * HALF, D)
    t2 = _mid_layer(acc1, W2, b1.reshape(1, D), deg_col)
    acc2 = edge_scatter(t2.reshape(N, 2, 128), ei).reshape(NC * HALF, D)
    return _final_layer(acc2, b2.reshape(1, D), deg_col)
